# Initial kernel scaffold; baseline (speedup 1.0000x reference)
import jax
import jax.numpy as jnp
from jax import lax
from jax.experimental import pallas as pl
from jax.experimental.pallas import tpu as pltpu
from jax.experimental.pallas import tpu_sc as plsc

N = 100000
E = 1600000
NP = 100352
HALFP = 50176
RPT = 3136
DPT = 6272
GB = 80
EROWS = E // GB
NSC = 2
NTILES = 16


def _rsqrt_nr(x):
  i = plsc.bitcast(x, jnp.int32)
  i = jnp.int32(0x5F3759DF) - lax.shift_right_logical(i, 1)
  y = plsc.bitcast(i, jnp.float32)
  for _ in range(3):
    y = y * (1.5 - 0.5 * x * y * y)
  return y


def _deg_body(dst_hbm, ew_hbm, deg_hbm, acc_sh, dstv, ewv, zv):
  c = lax.axis_index("c")
  t = lax.axis_index("s")

  def _zero(i, _):
    zv[pl.ds(i * 16, 16)] = jnp.zeros((16,), jnp.float32)
    return 0
  lax.fori_loop(0, DPT // 16, _zero, 0)
  pltpu.sync_copy(zv, acc_sh.at[pl.ds(t * DPT, DPT)])
  plsc.subcore_barrier()

  row0 = c * (EROWS // 2) + t * (EROWS // 32)

  def _chunk(i, _):
    r = row0 + i * 5
    pltpu.sync_copy(dst_hbm.at[pl.ds(r, 5)], dstv)
    pltpu.sync_copy(ew_hbm.at[pl.ds(r, 5)], ewv)
    for j in range(5):
      pltpu.sync_copy(ewv.at[j], acc_sh.at[dstv.at[j]], add=True)
    return 0
  lax.fori_loop(0, (EROWS // 32) // 5, _chunk, 0)
  plsc.subcore_barrier()
  pltpu.sync_copy(acc_sh.at[pl.ds(t * DPT, DPT)],
                  deg_hbm.at[c, pl.ds(t * DPT, DPT)])


def _deg_call(dst2, ew2):
  mesh = plsc.VectorSubcoreMesh(core_axis_name="c", subcore_axis_name="s")
  return pl.kernel(
      _deg_body,
      out_type=jax.ShapeDtypeStruct((NSC, NP), jnp.float32),
      mesh=mesh,
      scratch_types=[
          pltpu.VMEM_SHARED((NP,), jnp.float32),
          pltpu.VMEM((5, GB), jnp.int32),
          pltpu.VMEM((5, GB), jnp.float32),
          pltpu.VMEM((DPT,), jnp.float32),
      ],
  )(dst2, ew2)


def _zh_body(x_ref, wd_ref, bd_ref, wc_ref, d0_ref, d1_ref, zh_ref):
  xb = x_ref[...]
  h = lax.dot_general(xb, wd_ref[...], (((1,), (1,)), ((), ())),
                      preferred_element_type=jnp.float32)
  h = jnp.maximum(h + bd_ref[...], 0.0)
  z = lax.dot_general(h, wc_ref[...], (((1,), (1,)), ((), ())),
                      preferred_element_type=jnp.float32)
  dsum = d0_ref[...] + d1_ref[...] + 1.0
  zh_ref[...] = z * lax.rsqrt(dsum)


def _zh_call(x, wd, bd2, wc, d0c, d1c):
  blk = 1024
  grid = NP // blk
  return pl.pallas_call(
      _zh_body,
      grid=(grid,),
      in_specs=[
          pl.BlockSpec((blk, 128), lambda i: (i, 0)),
          pl.BlockSpec((32, 128), lambda i: (0, 0)),
          pl.BlockSpec((1, 32), lambda i: (0, 0)),
          pl.BlockSpec((32, 32), lambda i: (0, 0)),
          pl.BlockSpec((blk, 1), lambda i: (i, 0)),
          pl.BlockSpec((blk, 1), lambda i: (i, 0)),
      ],
      out_specs=pl.BlockSpec((blk, 32), lambda i: (i, 0)),
      out_shape=jax.ShapeDtypeStruct((NP, 32), jnp.float32),
  )(x, wd, bd2, wc, d0c, d1c)


def _agg_body(zh_hbm, src_hbm, dst_hbm, ew_hbm, deg_hbm, bc_hbm, out_hbm,
              acc_sh, srcv, dstv, ewv, lidx, rows, d0v, d1v, dinv,
              accv, outv, bcv, sem):
  c = lax.axis_index("c")
  t = lax.axis_index("s")
  g0 = c * HALFP + t * RPT

  pltpu.sync_copy(bc_hbm, bcv)
  pltpu.sync_copy(deg_hbm.at[0, pl.ds(g0, RPT)], d0v)
  pltpu.sync_copy(deg_hbm.at[1, pl.ds(g0, RPT)], d1v)

  def _nr(i, _):
    s = pl.ds(i * 16, 16)
    dinv[s] = _rsqrt_nr(d0v[s] + d1v[s] + 1.0)
    return 0
  lax.fori_loop(0, RPT // 16, _nr, 0)

  pltpu.sync_copy(zh_hbm.at[pl.ds(g0, RPT)], acc_sh.at[pl.ds(t * RPT, RPT)])
  plsc.subcore_barrier()

  base = c * HALFP
  row0 = t * (EROWS // NTILES)

  def _chunk(i, _):
    r = row0 + i * 5
    pltpu.sync_copy(src_hbm.at[pl.ds(r, 5)], srcv)
    pltpu.sync_copy(dst_hbm.at[pl.ds(r, 5)], dstv)
    pltpu.sync_copy(ew_hbm.at[pl.ds(r, 5)], ewv)
    for j in range(5):
      pltpu.async_copy(zh_hbm.at[srcv.at[j]], rows, sem).wait()

      def _group(g, _):
        gs = pl.ds(g * 16, 16)
        d16 = dstv[j, gs]
        l16 = d16 - base
        ok = (l16 >= 0) & (l16 < HALFP)
        lidx[j, gs] = jnp.where(ok, l16, HALFP)

        def _edge(e, _):
          idx = g * 16 + e
          w = ewv[j, idx]
          rows[idx, pl.ds(0, 16)] = rows[idx, pl.ds(0, 16)] * w
          rows[idx, pl.ds(16, 16)] = rows[idx, pl.ds(16, 16)] * w
          return 0
        lax.fori_loop(0, 16, _edge, 0)
        return 0
      lax.fori_loop(0, 5, _group, 0)
      pltpu.sync_copy(rows, acc_sh.at[lidx.at[j]], add=True)
    return 0
  lax.fori_loop(0, (EROWS // NTILES) // 5, _chunk, 0)
  plsc.subcore_barrier()

  b0 = bcv[pl.ds(0, 16)]
  b1 = bcv[pl.ds(16, 16)]
  for sc in range(8):
    r0 = sc * (RPT // 8)
    pltpu.sync_copy(acc_sh.at[pl.ds(t * RPT + r0, RPT // 8)], accv)

    def _fin(rr, _):
      dv = dinv[r0 + rr]
      outv[rr, pl.ds(0, 16)] = accv[rr, pl.ds(0, 16)] * dv + b0
      outv[rr, pl.ds(16, 16)] = accv[rr, pl.ds(16, 16)] * dv + b1
      return 0
    lax.fori_loop(0, RPT // 8, _fin, 0)
    pltpu.sync_copy(outv, out_hbm.at[pl.ds(g0 + r0, RPT // 8)])


def _agg_call(zh, src2, dst2, ew2, deg, bc):
  mesh = plsc.VectorSubcoreMesh(core_axis_name="c", subcore_axis_name="s")
  return pl.kernel(
      _agg_body,
      out_type=jax.ShapeDtypeStruct((NP, 32), jnp.float32),
      mesh=mesh,
      scratch_types=[
          pltpu.VMEM_SHARED((HALFP + 8, 32), jnp.float32),
          pltpu.VMEM((5, GB), jnp.int32),
          pltpu.VMEM((5, GB), jnp.int32),
          pltpu.VMEM((5, GB), jnp.float32),
          pltpu.VMEM((5, GB), jnp.int32),
          pltpu.VMEM((GB, 32), jnp.float32),
          pltpu.VMEM((RPT,), jnp.float32),
          pltpu.VMEM((RPT,), jnp.float32),
          pltpu.VMEM((RPT,), jnp.float32),
          pltpu.VMEM((RPT // 8, 32), jnp.float32),
          pltpu.VMEM((RPT // 8, 32), jnp.float32),
          pltpu.VMEM((32,), jnp.float32),
          pltpu.SemaphoreType.DMA,
      ],
  )(zh, src2, dst2, ew2, deg, bc)


@jax.jit
def kernel(x, edge_index, edge_attr, W_dense, b_dense, W_mu, b_mu,
           W_logstd, b_logstd):
  src2 = edge_index[0].reshape(EROWS, GB)
  dst2 = edge_index[1].reshape(EROWS, GB)
  ew2 = edge_attr.reshape(EROWS, GB)
  wc = jnp.concatenate([W_mu, W_logstd], axis=0)
  bc = jnp.concatenate([b_mu, b_logstd], axis=0)
  bd2 = b_dense.reshape(1, 32)

  deg = _deg_call(dst2, ew2)
  d0c = deg[0].reshape(NP, 1)
  d1c = deg[1].reshape(NP, 1)
  zh = _zh_call(x, W_dense, bd2, wc, d0c, d1c)
  out = _agg_call(zh, src2, dst2, ew2, deg, bc)
  return out[:N, :16], out[:N, 16:]


# speedup vs baseline: 16.2028x; 16.2028x over previous
import jax
import jax.numpy as jnp
from jax import lax
from jax.experimental import pallas as pl
from jax.experimental.pallas import tpu as pltpu
from jax.experimental.pallas import tpu_sc as plsc

N = 100000
E = 1600000
NP = 100352
HALFP = 50176
RPT = 3136
DPT = 6272
GB = 80
EROWS = 20480
EP = EROWS * GB
NSC = 2
NTILES = 16


def _rsqrt_nr(x):
  i = lax.bitcast_convert_type(x, jnp.int32)
  i = jnp.int32(0x5F3759DF) - lax.shift_right_logical(i, 1)
  y = lax.bitcast_convert_type(i, jnp.float32)
  for _ in range(3):
    y = y * (1.5 - 0.5 * x * y * y)
  return y


def _deg_body(dst_hbm, ew_hbm, deg0_hbm, deg1_hbm, acc_sh, dstv, ewv, zv):
  c = lax.axis_index("c")
  t = lax.axis_index("s")

  def _zero(i, _):
    zv[pl.ds(i * 16, 16)] = jnp.zeros((16,), jnp.float32)
    return 0
  lax.fori_loop(0, DPT // 16, _zero, 0)
  pltpu.sync_copy(zv, acc_sh.at[pl.ds(t * DPT, DPT)])
  plsc.subcore_barrier()

  row0 = c * (EROWS // 2) + t * (EROWS // 32)

  def _chunk(i, _):
    r = row0 + i * 8
    pltpu.sync_copy(dst_hbm.at[pl.ds(r, 8)], dstv)
    pltpu.sync_copy(ew_hbm.at[pl.ds(r, 8)], ewv)
    for j in range(8):
      pltpu.sync_copy(ewv.at[j], acc_sh.at[dstv.at[j]], add=True)
    return 0
  lax.fori_loop(0, (EROWS // 32) // 8, _chunk, 0)
  plsc.subcore_barrier()

  @pl.when(c == 0)
  def _():
    pltpu.sync_copy(acc_sh.at[pl.ds(t * DPT, DPT)],
                    deg0_hbm.at[pl.ds(t * DPT, DPT)])

  @pl.when(c == 1)
  def _():
    pltpu.sync_copy(acc_sh.at[pl.ds(t * DPT, DPT)],
                    deg1_hbm.at[pl.ds(t * DPT, DPT)])


def _deg_call(dst2, ew2):
  mesh = plsc.VectorSubcoreMesh(core_axis_name="c", subcore_axis_name="s")
  return pl.kernel(
      _deg_body,
      out_type=(jax.ShapeDtypeStruct((NP,), jnp.float32),
                jax.ShapeDtypeStruct((NP,), jnp.float32)),
      mesh=mesh,
      scratch_types=[
          pltpu.VMEM_SHARED((NP,), jnp.float32),
          pltpu.VMEM((8, GB), jnp.int32),
          pltpu.VMEM((8, GB), jnp.float32),
          pltpu.VMEM((DPT,), jnp.float32),
      ],
  )(dst2, ew2)


def _zh_body(x_ref, wd_ref, bd_ref, wc_ref, d0_ref, d1_ref, zh_ref):
  xb = x_ref[...]
  h = lax.dot_general(xb, wd_ref[...], (((1,), (1,)), ((), ())),
                      preferred_element_type=jnp.float32)
  h = jnp.maximum(h + bd_ref[...], 0.0)
  z = lax.dot_general(h, wc_ref[...], (((1,), (1,)), ((), ())),
                      preferred_element_type=jnp.float32)
  dsum = d0_ref[...] + d1_ref[...] + 1.0
  zh_ref[...] = z * lax.rsqrt(dsum)


def _zh_call(x, wd, bd2, wc, d0c, d1c):
  blk = 1024
  grid = NP // blk
  return pl.pallas_call(
      _zh_body,
      grid=(grid,),
      in_specs=[
          pl.BlockSpec((blk, 128), lambda i: (i, 0)),
          pl.BlockSpec((32, 128), lambda i: (0, 0)),
          pl.BlockSpec((1, 32), lambda i: (0, 0)),
          pl.BlockSpec((32, 32), lambda i: (0, 0)),
          pl.BlockSpec((blk, 1), lambda i: (i, 0)),
          pl.BlockSpec((blk, 1), lambda i: (i, 0)),
      ],
      out_specs=pl.BlockSpec((blk, 32), lambda i: (i, 0)),
      out_shape=jax.ShapeDtypeStruct((NP, 32), jnp.float32),
  )(x, wd, bd2, wc, d0c, d1c)


def _agg_body(zh_hbm, src_hbm, dst_hbm, ew_hbm, deg0_hbm, deg1_hbm, bc_hbm,
              out_hbm,
              acc_sh, srcv, dstv, ewv, lidx, rows, d0v, d1v, dinv,
              accv, outv, bcv, sem):
  c = lax.axis_index("c")
  t = lax.axis_index("s")
  g0 = c * HALFP + t * RPT

  pltpu.sync_copy(bc_hbm, bcv)
  pltpu.sync_copy(deg0_hbm.at[pl.ds(g0, RPT)], d0v)
  pltpu.sync_copy(deg1_hbm.at[pl.ds(g0, RPT)], d1v)

  def _nr(i, _):
    s = pl.ds(i * 16, 16)
    dinv[s] = _rsqrt_nr(d0v[s] + d1v[s] + 1.0)
    return 0
  lax.fori_loop(0, RPT // 16, _nr, 0)

  pltpu.sync_copy(zh_hbm.at[pl.ds(g0, RPT)], acc_sh.at[pl.ds(t * RPT, RPT)])
  plsc.subcore_barrier()

  base = c * HALFP
  row0 = t * (EROWS // NTILES)

  def _chunk(i, _):
    r = row0 + i * 8
    pltpu.sync_copy(src_hbm.at[pl.ds(r, 8)], srcv)
    pltpu.sync_copy(dst_hbm.at[pl.ds(r, 8)], dstv)
    pltpu.sync_copy(ew_hbm.at[pl.ds(r, 8)], ewv)
    for j in range(8):
      pltpu.async_copy(zh_hbm.at[srcv.at[j]], rows, sem).wait()

      def _group(g, _):
        gs = pl.ds(g * 16, 16)
        d16 = dstv[j, gs]
        l16 = d16 - base
        ok = (l16 >= 0) & (l16 < HALFP)
        lidx[j, gs] = jnp.where(ok, l16, HALFP)
        ew16 = ewv[j, gs]
        for e in range(16):
          idx = g * 16 + e
          w = ew16[e]
          rows[idx, pl.ds(0, 16)] = rows[idx, pl.ds(0, 16)] * w
          rows[idx, pl.ds(16, 16)] = rows[idx, pl.ds(16, 16)] * w
        return 0
      lax.fori_loop(0, 5, _group, 0)
      pltpu.sync_copy(rows, acc_sh.at[lidx.at[j]], add=True)
    return 0
  lax.fori_loop(0, (EROWS // NTILES) // 8, _chunk, 0)
  plsc.subcore_barrier()

  b0 = bcv[pl.ds(0, 16)]
  b1 = bcv[pl.ds(16, 16)]
  EPC = RPT // 28

  def _sub(sc, _):
    r0 = sc * EPC
    pltpu.sync_copy(acc_sh.at[pl.ds(t * RPT + r0, EPC)], accv)

    def _fin(g, _):
      dv16 = dinv[pl.ds(r0 + g * 16, 16)]
      for e in range(16):
        rr = g * 16 + e
        dv = dv16[e]
        outv[rr, pl.ds(0, 16)] = accv[rr, pl.ds(0, 16)] * dv + b0
        outv[rr, pl.ds(16, 16)] = accv[rr, pl.ds(16, 16)] * dv + b1
      return 0
    lax.fori_loop(0, EPC // 16, _fin, 0)
    pltpu.sync_copy(outv, out_hbm.at[pl.ds(g0 + r0, EPC)])
    return 0
  lax.fori_loop(0, 28, _sub, 0)


def _agg_call(zh, src2, dst2, ew2, deg0, deg1, bc):
  mesh = plsc.VectorSubcoreMesh(core_axis_name="c", subcore_axis_name="s")
  return pl.kernel(
      _agg_body,
      out_type=jax.ShapeDtypeStruct((NP, 32), jnp.float32),
      mesh=mesh,
      compiler_params=pltpu.CompilerParams(use_tc_tiling_on_sc=False),
      scratch_types=[
          pltpu.VMEM_SHARED((HALFP + 8, 32), jnp.float32),
          pltpu.VMEM((8, GB), jnp.int32),
          pltpu.VMEM((8, GB), jnp.int32),
          pltpu.VMEM((8, GB), jnp.float32),
          pltpu.VMEM((8, GB), jnp.int32),
          pltpu.VMEM((GB, 32), jnp.float32),
          pltpu.VMEM((RPT,), jnp.float32),
          pltpu.VMEM((RPT,), jnp.float32),
          pltpu.VMEM((RPT,), jnp.float32),
          pltpu.VMEM((RPT // 28, 32), jnp.float32),
          pltpu.VMEM((RPT // 28, 32), jnp.float32),
          pltpu.VMEM((32,), jnp.float32),
          pltpu.SemaphoreType.DMA,
      ],
  )(zh, src2, dst2, ew2, deg0, deg1, bc)


@jax.jit
def kernel(x, edge_index, edge_attr, W_dense, b_dense, W_mu, b_mu,
           W_logstd, b_logstd):
  zpad_i = jnp.zeros((EP - E,), jnp.int32)
  zpad_f = jnp.zeros((EP - E,), jnp.float32)
  src2 = jnp.concatenate([edge_index[0], zpad_i]).reshape(EROWS, GB)
  dst2 = jnp.concatenate([edge_index[1], zpad_i]).reshape(EROWS, GB)
  ew2 = jnp.concatenate([edge_attr, zpad_f]).reshape(EROWS, GB)
  wc = jnp.concatenate([W_mu, W_logstd], axis=0)
  bc = jnp.concatenate([b_mu, b_logstd], axis=0)
  bd2 = b_dense.reshape(1, 32)

  deg0, deg1 = _deg_call(dst2, ew2)
  d0c = deg0.reshape(NP, 1)
  d1c = deg1.reshape(NP, 1)
  zh = _zh_call(x, W_dense, bd2, wc, d0c, d1c)
  out = _agg_call(zh, src2, dst2, ew2, deg0, deg1, bc)
  return out[:N, :16], out[:N, 16:]


# speedup vs baseline: 20.3894x; 1.2584x over previous
import jax
import jax.numpy as jnp
from jax import lax
from jax.experimental import pallas as pl
from jax.experimental.pallas import tpu as pltpu
from jax.experimental.pallas import tpu_sc as plsc

N = 100000
E = 1600000
NP = 100352
HALFP = 50176
RPT = 3136
DPT = 6272
GB = 64
EROWS = 25600
EP = EROWS * GB
NSC = 2
NTILES = 16


def _rsqrt_nr(x):
  i = lax.bitcast_convert_type(x, jnp.int32)
  i = jnp.int32(0x5F3759DF) - lax.shift_right_logical(i, 1)
  y = lax.bitcast_convert_type(i, jnp.float32)
  for _ in range(3):
    y = y * (1.5 - 0.5 * x * y * y)
  return y


def _deg_body(dst_hbm, ew_hbm, deg0_hbm, deg1_hbm, acc_sh, dstv, ewv, zv):
  c = lax.axis_index("c")
  t = lax.axis_index("s")

  def _zero(i, _):
    zv[pl.ds(i * 16, 16)] = jnp.zeros((16,), jnp.float32)
    return 0
  lax.fori_loop(0, DPT // 16, _zero, 0)
  pltpu.sync_copy(zv, acc_sh.at[pl.ds(t * DPT, DPT)])
  plsc.subcore_barrier()

  row0 = c * (EROWS // 2) + t * (EROWS // 32)

  def _chunk(i, _):
    r = row0 + i * 8
    pltpu.sync_copy(dst_hbm.at[pl.ds(r, 8)], dstv)
    pltpu.sync_copy(ew_hbm.at[pl.ds(r, 8)], ewv)
    for j in range(8):
      pltpu.sync_copy(ewv.at[j], acc_sh.at[dstv.at[j]], add=True)
    return 0
  lax.fori_loop(0, (EROWS // 32) // 8, _chunk, 0)
  plsc.subcore_barrier()

  @pl.when(c == 0)
  def _():
    pltpu.sync_copy(acc_sh.at[pl.ds(t * DPT, DPT)],
                    deg0_hbm.at[pl.ds(t * DPT, DPT)])

  @pl.when(c == 1)
  def _():
    pltpu.sync_copy(acc_sh.at[pl.ds(t * DPT, DPT)],
                    deg1_hbm.at[pl.ds(t * DPT, DPT)])


def _deg_call(dst2, ew2):
  mesh = plsc.VectorSubcoreMesh(core_axis_name="c", subcore_axis_name="s")
  return pl.kernel(
      _deg_body,
      out_type=(jax.ShapeDtypeStruct((NP,), jnp.float32),
                jax.ShapeDtypeStruct((NP,), jnp.float32)),
      mesh=mesh,
      scratch_types=[
          pltpu.VMEM_SHARED((NP,), jnp.float32),
          pltpu.VMEM((8, GB), jnp.int32),
          pltpu.VMEM((8, GB), jnp.float32),
          pltpu.VMEM((DPT,), jnp.float32),
      ],
  )(dst2, ew2)


def _zh_body(x_ref, wd_ref, bd_ref, wc_ref, d0_ref, d1_ref, zh_ref):
  xb = x_ref[...]
  h = lax.dot_general(xb, wd_ref[...], (((1,), (1,)), ((), ())),
                      preferred_element_type=jnp.float32)
  h = jnp.maximum(h + bd_ref[...], 0.0)
  z = lax.dot_general(h, wc_ref[...], (((1,), (1,)), ((), ())),
                      preferred_element_type=jnp.float32)
  dsum = d0_ref[...] + d1_ref[...] + 1.0
  zh_ref[...] = z * lax.rsqrt(dsum)


def _zh_call(x, wd, bd2, wc, d0c, d1c):
  blk = 1024
  grid = NP // blk
  return pl.pallas_call(
      _zh_body,
      grid=(grid,),
      in_specs=[
          pl.BlockSpec((blk, 128), lambda i: (i, 0)),
          pl.BlockSpec((32, 128), lambda i: (0, 0)),
          pl.BlockSpec((1, 32), lambda i: (0, 0)),
          pl.BlockSpec((32, 32), lambda i: (0, 0)),
          pl.BlockSpec((blk, 1), lambda i: (i, 0)),
          pl.BlockSpec((blk, 1), lambda i: (i, 0)),
      ],
      out_specs=pl.BlockSpec((blk, 32), lambda i: (i, 0)),
      out_shape=jax.ShapeDtypeStruct((NP, 32), jnp.float32),
  )(x, wd, bd2, wc, d0c, d1c)


def _agg_body(zh_hbm, src_hbm, dst_hbm, ew_hbm, deg0_hbm, deg1_hbm, bc_hbm,
              out_hbm,
              acc_sh, srcv, dstv, ewv, lidx8, rows8, d0v, d1v, dinv,
              accv, outv, bcv, stsem0, stsem1, gsem, ssem):
  c = lax.axis_index("c")
  t = lax.axis_index("s")
  g0 = c * HALFP + t * RPT

  pltpu.sync_copy(bc_hbm, bcv)
  for nc in range(7):
    n0 = nc * 448
    pltpu.sync_copy(deg0_hbm.at[pl.ds(g0 + n0, 448)], d0v)
    pltpu.sync_copy(deg1_hbm.at[pl.ds(g0 + n0, 448)], d1v)

    def _nr2(i, _):
      so = pl.ds(n0 + i * 16, 16)
      si = pl.ds(i * 16, 16)
      dinv[so] = _rsqrt_nr(d0v[si] + d1v[si] + 1.0)
      return 0
    lax.fori_loop(0, 448 // 16, _nr2, 0)

  pltpu.sync_copy(zh_hbm.at[pl.ds(g0, RPT)], acc_sh.at[pl.ds(t * RPT, RPT)])
  plsc.subcore_barrier()

  base = c * HALFP
  row0 = t * (EROWS // NTILES)
  nbig = (EROWS // NTILES) // 8
  lastrow = row0 + (nbig - 1) * 8

  def _stage(st, brow):
    sem = stsem0 if st == 0 else stsem1
    pltpu.async_copy(src_hbm.at[pl.ds(brow, 8)], srcv.at[st], sem)
    pltpu.async_copy(dst_hbm.at[pl.ds(brow, 8)], dstv.at[st], sem)
    pltpu.async_copy(ew_hbm.at[pl.ds(brow, 8)], ewv.at[st], sem)

  def _drain_stage(st):
    sem = stsem0 if st == 0 else stsem1
    pltpu.make_async_copy(src_hbm.at[pl.ds(0, 8)], srcv.at[st], sem).wait()
    pltpu.make_async_copy(dst_hbm.at[pl.ds(0, 8)], dstv.at[st], sem).wait()
    pltpu.make_async_copy(ew_hbm.at[pl.ds(0, 8)], ewv.at[st], sem).wait()

  def _compute_batch(st, jj, k):
    def _group(g, _):
      gs = pl.ds(g * 16, 16)
      d16 = dstv[st, jj, gs]
      l16 = d16 - base
      ok = (l16 >= 0) & (l16 < HALFP)
      lidx8[k, gs] = jnp.where(ok, l16, HALFP)
      ew16 = ewv[st, jj, gs]
      for e in range(16):
        idx = g * 16 + e
        w = ew16[e]
        rows8[k, idx, pl.ds(0, 16)] = rows8[k, idx, pl.ds(0, 16)] * w
        rows8[k, idx, pl.ds(16, 16)] = rows8[k, idx, pl.ds(16, 16)] * w
      return 0
    lax.fori_loop(0, GB // 16, _group, 0)

  def _process(st):
    g1 = [pltpu.async_copy(zh_hbm.at[srcv.at[st, b]], rows8.at[b], gsem)
          for b in range(4)]
    g2 = [pltpu.async_copy(zh_hbm.at[srcv.at[st, 4 + b]], rows8.at[4 + b],
                           gsem) for b in range(4)]
    for ss in range(2):
      h = ss * 4
      for b in range(4):
        (g1 if ss == 0 else g2)[b].wait()
      for b in range(4):
        k = h + b
        _compute_batch(st, ss * 4 + b, k)
        pltpu.async_copy(rows8.at[k], acc_sh.at[lidx8.at[k]], ssem, add=True)
    for k in range(8):
      pltpu.make_async_copy(rows8.at[0], acc_sh.at[lidx8.at[0]], ssem).wait()

  _stage(0, row0)

  def _pair(i, _):
    brow_a = row0 + (2 * i) * 8
    _drain_stage(0)
    _stage(1, jnp.minimum(brow_a + 8, lastrow))
    _process(0)
    _drain_stage(1)
    _stage(0, jnp.minimum(brow_a + 16, lastrow))
    _process(1)
    return 0
  lax.fori_loop(0, nbig // 2, _pair, 0)
  _drain_stage(0)
  plsc.subcore_barrier()

  b0 = bcv[pl.ds(0, 16)]
  b1 = bcv[pl.ds(16, 16)]
  EPC = RPT // 49

  def _sub(sc, _):
    r0 = sc * EPC
    pltpu.sync_copy(acc_sh.at[pl.ds(t * RPT + r0, EPC)], accv)

    def _fin(g, _):
      dv16 = dinv[pl.ds(r0 + g * 16, 16)]
      for e in range(16):
        rr = g * 16 + e
        dv = dv16[e]
        outv[rr, pl.ds(0, 16)] = accv[rr, pl.ds(0, 16)] * dv + b0
        outv[rr, pl.ds(16, 16)] = accv[rr, pl.ds(16, 16)] * dv + b1
      return 0
    lax.fori_loop(0, EPC // 16, _fin, 0)
    pltpu.sync_copy(outv, out_hbm.at[pl.ds(g0 + r0, EPC)])
    return 0
  lax.fori_loop(0, 49, _sub, 0)


def _agg_call(zh, src2, dst2, ew2, deg0, deg1, bc):
  mesh = plsc.VectorSubcoreMesh(core_axis_name="c", subcore_axis_name="s")
  return pl.kernel(
      _agg_body,
      out_type=jax.ShapeDtypeStruct((NP, 32), jnp.float32),
      mesh=mesh,
      compiler_params=pltpu.CompilerParams(use_tc_tiling_on_sc=False),
      scratch_types=[
          pltpu.VMEM_SHARED((HALFP + 1, 32), jnp.float32),
          pltpu.VMEM((2, 8, GB), jnp.int32),
          pltpu.VMEM((2, 8, GB), jnp.int32),
          pltpu.VMEM((2, 8, GB), jnp.float32),
          pltpu.VMEM((8, GB), jnp.int32),
          pltpu.VMEM((8, GB, 32), jnp.float32),
          pltpu.VMEM((448,), jnp.float32),
          pltpu.VMEM((448,), jnp.float32),
          pltpu.VMEM((RPT,), jnp.float32),
          pltpu.VMEM((RPT // 49, 32), jnp.float32),
          pltpu.VMEM((RPT // 49, 32), jnp.float32),
          pltpu.VMEM((32,), jnp.float32),
          pltpu.SemaphoreType.DMA,
          pltpu.SemaphoreType.DMA,
          pltpu.SemaphoreType.DMA,
          pltpu.SemaphoreType.DMA,
      ],
  )(zh, src2, dst2, ew2, deg0, deg1, bc)


@jax.jit
def kernel(x, edge_index, edge_attr, W_dense, b_dense, W_mu, b_mu,
           W_logstd, b_logstd):
  zpad_i = jnp.zeros((EP - E,), jnp.int32)
  zpad_f = jnp.zeros((EP - E,), jnp.float32)
  src2 = jnp.concatenate([edge_index[0], zpad_i]).reshape(EROWS, GB)
  dst2 = jnp.concatenate([edge_index[1], zpad_i]).reshape(EROWS, GB)
  ew2 = jnp.concatenate([edge_attr, zpad_f]).reshape(EROWS, GB)
  wc = jnp.concatenate([W_mu, W_logstd], axis=0)
  bc = jnp.concatenate([b_mu, b_logstd], axis=0)
  bd2 = b_dense.reshape(1, 32)

  deg0, deg1 = _deg_call(dst2, ew2)
  d0c = deg0.reshape(NP, 1)
  d1c = deg1.reshape(NP, 1)
  zh = _zh_call(x, W_dense, bd2, wc, d0c, d1c)
  out = _agg_call(zh, src2, dst2, ew2, deg0, deg1, bc)
  return out[:N, :16], out[:N, 16:]
